# baseline (speedup 1.0000x reference)
import functools

import jax
import jax.numpy as jnp
from jax import lax
from jax.experimental import pallas as pl
from jax.experimental.pallas import tpu as pltpu
from jax.experimental.pallas import tpu_sc as plsc

_NC, _NS, _L = 2, 16, 16
_NW = _NC * _NS


def _proj_body(emb_ref, w_ref, t_ref):
    t_ref[...] = jnp.sum(emb_ref[...] * w_ref[...], axis=-1)


def _project_table(emb_table, w_row):
    V, D = emb_table.shape
    G, rb = 20, 625
    emb4 = emb_table.reshape(G, rb, 8, D)
    w4 = w_row.reshape(1, 1, 1, D)
    t3 = pl.pallas_call(
        _proj_body,
        grid=(G,),
        in_specs=[
            pl.BlockSpec((1, rb, 8, D), lambda i: (i, 0, 0, 0)),
            pl.BlockSpec((1, 1, 1, D), lambda i: (0, 0, 0, 0)),
        ],
        out_specs=pl.BlockSpec((1, rb, 8), lambda i: (i, 0, 0)),
        out_shape=jax.ShapeDtypeStruct((G, rb, 8), jnp.float32),
    )(emb4, w4)
    return t3.reshape(V)


def _make_sc_sum(V, S, rpt):
    nchunk = rpt // _L
    mesh = plsc.VectorSubcoreMesh(core_axis_name="c", subcore_axis_name="s")

    @functools.partial(
        pl.kernel,
        mesh=mesh,
        out_type=jax.ShapeDtypeStruct((_NW, rpt), jnp.float32),
        scratch_types=[
            pltpu.VMEM((V,), jnp.float32),
            pltpu.VMEM((rpt * S,), jnp.int32),
            pltpu.VMEM((rpt,), jnp.int32),
            pltpu.VMEM((rpt,), jnp.float32),
            pltpu.SemaphoreType.DMA,
            pltpu.SemaphoreType.DMA,
        ],
        compiler_params=pltpu.CompilerParams(needs_layout_passes=False),
    )
    def sc_sum(t_hbm, cap_hbm, len_hbm, out_hbm, t_v, cap_v, len_v, o_v, sem_t, sem_c):
        wid = lax.axis_index("s") * _NC + lax.axis_index("c")
        cp_t = pltpu.async_copy(t_hbm, t_v, sem_t)
        cp_c = pltpu.async_copy(cap_hbm.at[wid], cap_v, sem_c)
        pltpu.sync_copy(len_hbm.at[wid], len_v)
        cp_c.wait()
        cp_t.wait()

        lanes = lax.iota(jnp.int32, _L)

        def body(c, carry):
            base = c * _L
            rowb = (base + lanes) * S
            l16 = len_v[pl.ds(base, _L)]
            acc = jnp.zeros((_L,), jnp.float32)
            for j in range(S):
                idx = plsc.load_gather(cap_v, [rowb + j])
                idxm = jnp.where(l16 > (j + 1), idx, 0)
                acc = acc + plsc.load_gather(t_v, [idxm])
            o_v[pl.ds(base, _L)] = acc
            return carry

        lax.fori_loop(0, nchunk, body, 0)
        pltpu.sync_copy(o_v, out_hbm.at[wid])

    return sc_sum


def _post_body(s_ref, dfall_ref, idall_ref, ix_ref, b_ref, out_ref):
    idf = dfall_ref[...] * (idall_ref[...] == ix_ref[...]).astype(jnp.float32)
    sc = (s_ref[...] + b_ref[0, 0]) * idf
    m = jnp.max(sc, axis=1, keepdims=True)
    e = jnp.exp(sc - m)
    lse = jnp.log(jnp.sum(e, axis=1, keepdims=True)) + m
    out_ref[...] = sc - lse


def kernel(captions, caption_lengths, logs, idall, dfall, ix, emb_table, W_out, b_out):
    del logs
    B, BEAM, S = captions.shape
    V, D = emb_table.shape
    R = B * BEAM
    rpt = R // _NW

    t = _project_table(emb_table, W_out[0])

    cap_t = captions.reshape(_NW, rpt * S)
    len_t = caption_lengths.reshape(_NW, rpt)

    sraw = _make_sc_sum(V, S, rpt)(t, cap_t, len_t).reshape(B, BEAM)

    out = pl.pallas_call(
        _post_body,
        out_shape=jax.ShapeDtypeStruct((B, BEAM), jnp.float32),
    )(sraw, dfall, idall, ix.reshape(B, 1), b_out.reshape(1, 1))
    return out


# speedup vs baseline: 1.0978x; 1.0978x over previous
import functools

import jax
import jax.numpy as jnp
from jax import lax
from jax.experimental import pallas as pl
from jax.experimental.pallas import tpu as pltpu
from jax.experimental.pallas import tpu_sc as plsc

_NC, _NS, _L = 2, 16, 16
_NW = _NC * _NS


_TROWS = 784


def _proj_body(emb_ref, w_ref, t_ref):
    rows = t_ref.shape[0]
    x3 = emb_ref[...].reshape(rows, 128, 128)
    w2 = w_ref[...]
    for a in range(rows):
        t_ref[a : a + 1, :] = jax.lax.dot_general(
            w2, x3[a], (((1,), (1,)), ((), ()))
        )


def _project_table(emb_table, w_row):
    V, D = emb_table.shape
    grid = 14
    bo = _TROWS // grid
    t2 = pl.pallas_call(
        _proj_body,
        grid=(grid,),
        in_specs=[
            pl.BlockSpec((bo * 128, D), lambda i: (i, 0)),
            pl.BlockSpec((1, D), lambda i: (0, 0)),
        ],
        out_specs=pl.BlockSpec((bo, 128), lambda i: (i, 0)),
        out_shape=jax.ShapeDtypeStruct((_TROWS, 128), jnp.float32),
    )(emb_table, w_row.reshape(1, D))
    return t2.reshape(_TROWS * 128)


def _make_sc_sum(Vp, S, rpt):
    nchunk = rpt // _L
    mesh = plsc.VectorSubcoreMesh(core_axis_name="c", subcore_axis_name="s")

    @functools.partial(
        pl.kernel,
        mesh=mesh,
        out_type=jax.ShapeDtypeStruct((_NW, rpt), jnp.float32),
        scratch_types=[
            pltpu.VMEM((Vp,), jnp.float32),
            pltpu.VMEM((rpt * S,), jnp.int32),
            pltpu.VMEM((rpt,), jnp.int32),
            pltpu.VMEM((rpt,), jnp.float32),
            pltpu.SemaphoreType.DMA,
            pltpu.SemaphoreType.DMA,
        ],
        compiler_params=pltpu.CompilerParams(needs_layout_passes=False),
    )
    def sc_sum(t_hbm, cap_hbm, len_hbm, out_hbm, t_v, cap_v, len_v, o_v, sem_t, sem_c):
        wid = lax.axis_index("s") * _NC + lax.axis_index("c")
        cp_t = pltpu.async_copy(t_hbm, t_v, sem_t)
        cp_c = pltpu.async_copy(cap_hbm.at[wid], cap_v, sem_c)
        pltpu.sync_copy(len_hbm.at[wid], len_v)
        cp_c.wait()
        cp_t.wait()

        lanesS = lax.iota(jnp.int32, _L) * S

        def body(c, carry):
            base = c * _L
            rowb = lanesS + (base * S)
            l16 = len_v[pl.ds(base, _L)]
            acc = jnp.zeros((_L,), jnp.float32)
            for j in range(S):
                idx = plsc.load_gather(cap_v, [rowb + j])
                idxm = jnp.where(l16 > (j + 1), idx, 0)
                acc = acc + plsc.load_gather(t_v, [idxm])
            o_v[pl.ds(base, _L)] = acc
            return carry

        lax.fori_loop(0, nchunk, body, 0)
        pltpu.sync_copy(o_v, out_hbm.at[wid])

    return sc_sum


def _post_body(s_ref, dfall_ref, idall_ref, ix_ref, b_ref, out_ref):
    idf = dfall_ref[...] * (idall_ref[...] == ix_ref[...]).astype(jnp.float32)
    sc = (s_ref[...] + b_ref[0, 0]) * idf
    m = jnp.max(sc, axis=1, keepdims=True)
    e = jnp.exp(sc - m)
    lse = jnp.log(jnp.sum(e, axis=1, keepdims=True)) + m
    out_ref[...] = sc - lse


def kernel(captions, caption_lengths, logs, idall, dfall, ix, emb_table, W_out, b_out):
    del logs
    B, BEAM, S = captions.shape
    V, D = emb_table.shape
    R = B * BEAM
    rpt = R // _NW

    t = _project_table(emb_table, W_out[0])

    cap_t = captions.reshape(_NW, rpt * S)
    len_t = caption_lengths.reshape(_NW, rpt)

    sraw = _make_sc_sum(_TROWS * 128, S, rpt)(t, cap_t, len_t).reshape(B, BEAM)

    out = pl.pallas_call(
        _post_body,
        out_shape=jax.ShapeDtypeStruct((B, BEAM), jnp.float32),
    )(sraw, dfall, idall, ix.reshape(B, 1), b_out.reshape(1, 1))
    return out


# speedup vs baseline: 1.7919x; 1.6323x over previous
import functools

import jax
import jax.numpy as jnp
from jax import lax
from jax.experimental import pallas as pl
from jax.experimental.pallas import tpu as pltpu
from jax.experimental.pallas import tpu_sc as plsc

_NC, _NS, _L = 2, 16, 16
_NW = _NC * _NS


_TROWS = 784


def _proj_body(emb_ref, w_ref, t_ref):
    rows = t_ref.shape[0]
    x3 = emb_ref[...].reshape(rows, 128, 128)
    w2 = w_ref[...]
    for a in range(rows):
        t_ref[a : a + 1, :] = jax.lax.dot_general(
            w2, x3[a], (((1,), (1,)), ((), ()))
        )


def _project_table(emb_table, w_row):
    V, D = emb_table.shape
    grid = 7
    bo = _TROWS // grid
    t2 = pl.pallas_call(
        _proj_body,
        grid=(grid,),
        in_specs=[
            pl.BlockSpec((bo * 128, D), lambda i: (i, 0)),
            pl.BlockSpec((1, D), lambda i: (0, 0)),
        ],
        out_specs=pl.BlockSpec((bo, 128), lambda i: (i, 0)),
        out_shape=jax.ShapeDtypeStruct((_TROWS, 128), jnp.float32),
    )(emb_table, w_row.reshape(1, D))
    return t2.reshape(_TROWS * 128)


def _make_sc_sum(Vp, B, BEAM, S):
    tiles_per_beam = _NW // BEAM
    bpt = B // tiles_per_beam
    nchunk = bpt // _L
    mesh = plsc.VectorSubcoreMesh(core_axis_name="c", subcore_axis_name="s")

    @functools.partial(
        pl.kernel,
        mesh=mesh,
        out_type=jax.ShapeDtypeStruct((BEAM, B), jnp.float32),
        scratch_types=[
            pltpu.VMEM((Vp,), jnp.float32),
            pltpu.VMEM((S, bpt), jnp.int32),
            pltpu.VMEM((bpt,), jnp.int32),
            pltpu.VMEM((bpt,), jnp.float32),
            pltpu.SemaphoreType.DMA,
            pltpu.SemaphoreType.DMA,
        ],
        compiler_params=pltpu.CompilerParams(needs_layout_passes=False),
    )
    def sc_sum(t_hbm, cap_hbm, len_hbm, out_hbm, t_v, cap_v, len_v, o_v, sem_t, sem_c):
        wid = lax.axis_index("s") * _NC + lax.axis_index("c")
        beam = wid // tiles_per_beam
        b0 = (wid % tiles_per_beam) * bpt
        cp_t = pltpu.async_copy(t_hbm, t_v, sem_t)
        cap_cps = [
            pltpu.async_copy(
                cap_hbm.at[beam * S + j, pl.ds(b0, bpt)], cap_v.at[j], sem_c
            )
            for j in range(S)
        ]
        pltpu.sync_copy(len_hbm.at[beam, pl.ds(b0, bpt)], len_v)
        for cp in cap_cps:
            cp.wait()

        def mask_body(c, carry):
            base = c * _L
            l16 = len_v[pl.ds(base, _L)]
            for j in range(S):
                idx = cap_v[j, pl.ds(base, _L)]
                cap_v[j, pl.ds(base, _L)] = jnp.where(l16 > (j + 1), idx, 0)
            return carry

        lax.fori_loop(0, nchunk, mask_body, 0)
        cp_t.wait()

        def body(c, carry):
            base = c * _L
            acc = jnp.zeros((_L,), jnp.float32)
            for j in range(S):
                acc = acc + plsc.load_gather(t_v, [cap_v[j, pl.ds(base, _L)]])
            o_v[pl.ds(base, _L)] = acc
            return carry

        lax.fori_loop(0, nchunk, body, 0)
        pltpu.sync_copy(o_v, out_hbm.at[beam, pl.ds(b0, bpt)])

    return sc_sum


def _post_body(s_ref, dfall_ref, idall_ref, ix_ref, b_ref, out_ref):
    idf = dfall_ref[...] * (idall_ref[...] == ix_ref[...]).astype(jnp.float32)
    sc = (s_ref[...] + b_ref[0, 0]) * idf
    m = jnp.max(sc, axis=0, keepdims=True)
    e = jnp.exp(sc - m)
    lse = jnp.log(jnp.sum(e, axis=0, keepdims=True)) + m
    out_ref[...] = sc - lse


def kernel(captions, caption_lengths, logs, idall, dfall, ix, emb_table, W_out, b_out):
    del logs
    B, BEAM, S = captions.shape
    V, D = emb_table.shape
    t = _project_table(emb_table, W_out[0])

    perm = jnp.eye(BEAM * S, dtype=jnp.float32).reshape(BEAM * S, BEAM, S)
    cap_k = jnp.einsum(
        "kmj,bmj->kb",
        perm,
        captions.astype(jnp.float32),
        precision=jax.lax.Precision.HIGHEST,
    ).astype(jnp.int32)
    cap_t = cap_k
    len_t = caption_lengths.reshape(B, BEAM).T

    sraw = _make_sc_sum(_TROWS * 128, B, BEAM, S)(t, cap_t, len_t)

    out_t = pl.pallas_call(
        _post_body,
        out_shape=jax.ShapeDtypeStruct((BEAM, B), jnp.float32),
    )(sraw, dfall.T, idall.T, ix.reshape(1, B), b_out.reshape(1, 1))
    return out_t.T
